# baseline (speedup 1.0000x reference)
import functools

import jax
import jax.numpy as jnp
from jax import lax
from jax.experimental import pallas as pl
from jax.experimental.pallas import tpu as pltpu
from jax.experimental.pallas import tpu_sc as plsc

B = 16384
V = 100000
R = 64
NC = 2
NS = 16
NW = NC * NS
BPW = B // NW
CH = 128
W = 4096
GRID_T = 7
S2 = W * GRID_T
LASTB = (V - 1) // W


def _pack_pairs(t):
    lo = t[:, 0:32].astype(jnp.bfloat16).astype(jnp.float32)
    hi = t[:, 32:64].astype(jnp.bfloat16).astype(jnp.float32)
    lo_u = lax.bitcast_convert_type(lo, jnp.uint32) >> 16
    hi_u = lax.bitcast_convert_type(hi, jnp.uint32) & jnp.uint32(0xFFFF0000)
    return lax.bitcast_convert_type(lo_u | hi_u, jnp.int32)


def _table_prep_body(xa_ref, xb_ref, xc_ref, xd_ref, o_ref):
    eye = (lax.broadcasted_iota(jnp.int32, (R, R), 0)
           == lax.broadcasted_iota(jnp.int32, (R, R), 1)).astype(jnp.float32)
    dn = (((0,), (0,)), ((), ()))
    packed = [
        _pack_pairs(lax.dot_general(x_ref[:], eye, dn,
                                    preferred_element_type=jnp.float32))
        for x_ref in (xa_ref, xb_ref, xc_ref, xd_ref)
    ]
    o_ref[:] = jnp.concatenate(packed, axis=1)


def _idx_prep_body(x_ref, o_ref):
    o_ref[:] = x_ref[:].reshape(3 * B)


MASK_HI = -65536


def _cpd_sc_body(idx_hbm, t0_hbm, t1_hbm, t2_hbm, out_hbm,
                 idx_v, hidx_v, r0, r1, r2, out_v, sem):
    wid = lax.axis_index("s") * NC + lax.axis_index("c")
    base = wid * BPW

    for m in range(3):
        pltpu.sync_copy(idx_hbm.at[pl.ds(m * B + base, BPW)], idx_v.at[m])
    for m in range(3):
        for k in range(BPW // 16):
            iv = idx_v[m, pl.ds(k * 16, 16)]
            q = (jnp.where(iv >= S2, 1, 0)
                 + jnp.where(iv >= 2 * S2, 1, 0)
                 + jnp.where(iv >= 3 * S2, 1, 0))
            hidx_v[m, pl.ds(k * 16, 16)] = 4 * (iv - q * S2) + q

    copies = []
    for m, (tab, dst) in enumerate(((t0_hbm, r0), (t1_hbm, r1),
                                    (t2_hbm, r2))):
        for j in range(BPW // CH):
            copies.append(pltpu.async_copy(
                tab.at[hidx_v.at[m, pl.ds(j * CH, CH)]],
                dst.at[pl.ds(j * CH, CH)], sem))
    for cp in copies:
        cp.wait()

    def group(g, c2):
        b0 = g * 16
        for rr in range(16):
            row = b0 + rr
            acc = None
            for c in range(2):
                wa = r0[row, pl.ds(c * 16, 16)]
                wb = r1[row, pl.ds(c * 16, 16)]
                wd = r2[row, pl.ds(c * 16, 16)]
                alo = plsc.bitcast(wa << 16, jnp.float32)
                blo = plsc.bitcast(wb << 16, jnp.float32)
                dlo = plsc.bitcast(wd << 16, jnp.float32)
                ahi = plsc.bitcast(wa & MASK_HI, jnp.float32)
                bhi = plsc.bitcast(wb & MASK_HI, jnp.float32)
                dhi = plsc.bitcast(wd & MASK_HI, jnp.float32)
                pv = alo * blo * dlo + ahi * bhi * dhi
                acc = pv if acc is None else acc + pv
            out_v[pl.ds(row * 16, 16)] = acc
        return c2

    lax.fori_loop(0, BPW // 16, group, 0)

    pltpu.sync_copy(out_v, out_hbm.at[pl.ds(wid * BPW * 16, BPW * 16)])


def _reduce_tc_body(x_ref, o_ref):
    x = x_ref[:].reshape(256, 128)
    l_i = lax.broadcasted_iota(jnp.int32, (128, 128), 0)
    j_i = lax.broadcasted_iota(jnp.int32, (128, 128), 1)
    wa = ((j_i < 64) & (l_i // 2 == j_i)).astype(jnp.float32)
    wb = ((j_i >= 64) & (l_i // 2 == j_i - 64)).astype(jnp.float32)
    n = 256
    for _ in range(4):
        h = n // 2
        r_i = lax.broadcasted_iota(jnp.int32, (h, n), 0)
        c_i = lax.broadcasted_iota(jnp.int32, (h, n), 1)
        ae = (c_i == 2 * r_i).astype(jnp.float32)
        ao = (c_i == 2 * r_i + 1).astype(jnp.float32)
        xe = jnp.dot(ae, x, preferred_element_type=jnp.float32)
        xo = jnp.dot(ao, x, preferred_element_type=jnp.float32)
        x = (jnp.dot(xe, wa, preferred_element_type=jnp.float32)
             + jnp.dot(xo, wb, preferred_element_type=jnp.float32))
        n = h
    o_ref[:] = x.reshape(2048)


def _prep_table(ev):
    return pl.pallas_call(
        _table_prep_body,
        grid=(GRID_T,),
        in_specs=[
            pl.BlockSpec((R, W), lambda g: (0, g)),
            pl.BlockSpec((R, W), lambda g: (0, g + GRID_T)),
            pl.BlockSpec((R, W), lambda g: (0, g + 2 * GRID_T)),
            pl.BlockSpec((R, W),
                         lambda g: (0, jnp.minimum(g + 3 * GRID_T, LASTB))),
        ],
        out_specs=pl.BlockSpec((W, 2 * R), lambda g: (g, 0)),
        out_shape=jax.ShapeDtypeStruct((S2, 2 * R), jnp.int32),
    )(ev, ev, ev, ev)


@jax.jit
def kernel(idxs, E0, E1, E2):
    idxs = idxs.astype(jnp.int32)

    idx_flat = pl.pallas_call(
        _idx_prep_body,
        in_specs=[pl.BlockSpec((3, B), lambda: (0, 0))],
        out_specs=pl.BlockSpec((3 * B,), lambda: (0,)),
        out_shape=jax.ShapeDtypeStruct((3 * B,), jnp.int32),
    )(jnp.transpose(idxs))

    t0 = _prep_table(jnp.transpose(E0)).reshape(4 * S2, 32)
    t1 = _prep_table(jnp.transpose(E1)).reshape(4 * S2, 32)
    t2 = _prep_table(jnp.transpose(E2)).reshape(4 * S2, 32)

    mesh = plsc.VectorSubcoreMesh(core_axis_name="c", subcore_axis_name="s")
    sc_fn = pl.kernel(
        _cpd_sc_body,
        mesh=mesh,
        out_type=jax.ShapeDtypeStruct((B * 16,), jnp.float32),
        scratch_types=[
            pltpu.VMEM((3, BPW), jnp.int32),
            pltpu.VMEM((3, BPW), jnp.int32),
            pltpu.VMEM((BPW, 32), jnp.int32),
            pltpu.VMEM((BPW, 32), jnp.int32),
            pltpu.VMEM((BPW, 32), jnp.int32),
            pltpu.VMEM((BPW * 16,), jnp.float32),
            pltpu.SemaphoreType.DMA,
        ],
        compiler_params=pltpu.CompilerParams(use_tc_tiling_on_sc=False, needs_layout_passes=False),
    )
    partials = sc_fn(idx_flat, t0, t1, t2)

    red_rows = 2048
    out = pl.pallas_call(
        _reduce_tc_body,
        grid=(B // red_rows,),
        in_specs=[pl.BlockSpec((red_rows * 16,), lambda i: (i,))],
        out_specs=pl.BlockSpec((red_rows,), lambda i: (i,)),
        out_shape=jax.ShapeDtypeStruct((B,), jnp.float32),
    )(partials)
    return out


# speedup vs baseline: 1.2470x; 1.2470x over previous
import functools

import jax
import jax.numpy as jnp
from jax import lax
from jax.experimental import pallas as pl
from jax.experimental.pallas import tpu as pltpu
from jax.experimental.pallas import tpu_sc as plsc

B = 16384
V = 100000
SPLIT = 57344
R = 64
NC = 2
NS = 16
NW = NC * NS
BPW = B // NW
CH = 128
PASS_ROWS = 256
W = 8192
GRID_T = SPLIT // W


def _table_prep_body(xa_ref, xb_ref, o_ref):
    eye = (lax.broadcasted_iota(jnp.int32, (R, R), 0)
           == lax.broadcasted_iota(jnp.int32, (R, R), 1)).astype(jnp.float32)
    dn = (((0,), (0,)), ((), ()))
    ta = lax.dot_general(xa_ref[:], eye, dn,
                         preferred_element_type=jnp.float32)
    tb = lax.dot_general(xb_ref[:], eye, dn,
                         preferred_element_type=jnp.float32)
    o_ref[:] = jnp.concatenate([ta, tb], axis=1)


def _idx_prep_body(x_ref, o_ref):
    o_ref[:] = x_ref[:].reshape(3 * B)


def _cpd_sc_body(idx_hbm, t0_hbm, t1_hbm, t2_hbm, out_hbm,
                 idx_v, hidx_v, r0, r1, r2, out_v, sem):
    wid = lax.axis_index("s") * NC + lax.axis_index("c")
    base = wid * BPW

    for m in range(3):
        pltpu.sync_copy(idx_hbm.at[pl.ds(m * B + base, BPW)], idx_v.at[m])
    for m in range(3):
        for k in range(BPW // 16):
            iv = idx_v[m, pl.ds(k * 16, 16)]
            hidx_v[m, pl.ds(k * 16, 16)] = jnp.where(
                iv >= SPLIT, 2 * (iv - SPLIT) + 1, 2 * iv)

    copies = []
    for m, (tab, dst) in enumerate(((t0_hbm, r0), (t1_hbm, r1),
                                    (t2_hbm, r2))):
        for j in range(BPW // CH):
            copies.append(pltpu.async_copy(
                tab.at[hidx_v.at[m, pl.ds(j * CH, CH)]],
                dst.at[pl.ds(j * CH, CH)], sem))
    for cp in copies:
        cp.wait()

    def group(g, c2):
        b0 = g * 16
        for rr in range(16):
            row = b0 + rr
            acc = None
            for c in range(4):
                a = r0[row, pl.ds(c * 16, 16)]
                bb = r1[row, pl.ds(c * 16, 16)]
                d = r2[row, pl.ds(c * 16, 16)]
                pv = a * bb * d
                acc = pv if acc is None else acc + pv
            out_v[pl.ds(row * 16, 16)] = acc
        return c2

    lax.fori_loop(0, BPW // 16, group, 0)

    pltpu.sync_copy(out_v, out_hbm.at[pl.ds(wid * BPW * 16, BPW * 16)])


def _reduce_tc_body(x_ref, o_ref):
    x = x_ref[:].reshape(256, 128)
    l_i = lax.broadcasted_iota(jnp.int32, (128, 128), 0)
    j_i = lax.broadcasted_iota(jnp.int32, (128, 128), 1)
    wa = ((j_i < 64) & (l_i // 2 == j_i)).astype(jnp.float32)
    wb = ((j_i >= 64) & (l_i // 2 == j_i - 64)).astype(jnp.float32)
    n = 256
    for _ in range(4):
        h = n // 2
        r_i = lax.broadcasted_iota(jnp.int32, (h, n), 0)
        c_i = lax.broadcasted_iota(jnp.int32, (h, n), 1)
        ae = (c_i == 2 * r_i).astype(jnp.float32)
        ao = (c_i == 2 * r_i + 1).astype(jnp.float32)
        xe = jnp.dot(ae, x, preferred_element_type=jnp.float32)
        xo = jnp.dot(ao, x, preferred_element_type=jnp.float32)
        x = (jnp.dot(xe, wa, preferred_element_type=jnp.float32)
             + jnp.dot(xo, wb, preferred_element_type=jnp.float32))
        n = h
    o_ref[:] = x.reshape(2048)


def _prep_table(ev):
    return pl.pallas_call(
        _table_prep_body,
        grid=(GRID_T,),
        in_specs=[pl.BlockSpec((R, W), lambda g: (0, g)),
                  pl.BlockSpec((R, W),
                               lambda g: (0, jnp.minimum(g + GRID_T,
                                                         (V - 1) // W)))],
        out_specs=pl.BlockSpec((W, 2 * R), lambda g: (g, 0)),
        out_shape=jax.ShapeDtypeStruct((SPLIT, 2 * R), jnp.float32),
    )(ev, ev)


@jax.jit
def kernel(idxs, E0, E1, E2):
    idxs = idxs.astype(jnp.int32)

    idx_flat = pl.pallas_call(
        _idx_prep_body,
        in_specs=[pl.BlockSpec((3, B), lambda: (0, 0))],
        out_specs=pl.BlockSpec((3 * B,), lambda: (0,)),
        out_shape=jax.ShapeDtypeStruct((3 * B,), jnp.int32),
    )(jnp.transpose(idxs))

    t0 = _prep_table(jnp.transpose(E0)).reshape(2 * SPLIT, R)
    t1 = _prep_table(jnp.transpose(E1)).reshape(2 * SPLIT, R)
    t2 = _prep_table(jnp.transpose(E2)).reshape(2 * SPLIT, R)

    mesh = plsc.VectorSubcoreMesh(core_axis_name="c", subcore_axis_name="s")
    sc_fn = pl.kernel(
        _cpd_sc_body,
        mesh=mesh,
        out_type=jax.ShapeDtypeStruct((B * 16,), jnp.float32),
        scratch_types=[
            pltpu.VMEM((3, BPW), jnp.int32),
            pltpu.VMEM((3, BPW), jnp.int32),
            pltpu.VMEM((BPW, R), jnp.float32),
            pltpu.VMEM((BPW, R), jnp.float32),
            pltpu.VMEM((BPW, R), jnp.float32),
            pltpu.VMEM((BPW * 16,), jnp.float32),
            pltpu.SemaphoreType.DMA,
        ],
        compiler_params=pltpu.CompilerParams(use_tc_tiling_on_sc=False),
    )
    partials = sc_fn(idx_flat, t0, t1, t2)

    red_rows = 2048
    out = pl.pallas_call(
        _reduce_tc_body,
        grid=(B // red_rows,),
        in_specs=[pl.BlockSpec((red_rows * 16,), lambda i: (i,))],
        out_specs=pl.BlockSpec((red_rows,), lambda i: (i,)),
        out_shape=jax.ShapeDtypeStruct((B,), jnp.float32),
    )(partials)
    return out


# speedup vs baseline: 1.2761x; 1.0233x over previous
import functools

import jax
import jax.numpy as jnp
from jax import lax
from jax.experimental import pallas as pl
from jax.experimental.pallas import tpu as pltpu
from jax.experimental.pallas import tpu_sc as plsc

B = 16384
V = 100000
SPLIT = 57344
R = 64
NC = 2
NS = 16
NW = NC * NS
BPW = B // NW
CH = 128
PASS_ROWS = 256
W = 8192
GRID_T = SPLIT // W


def _table_prep_body(xa_ref, xb_ref, o_ref):
    eye = (lax.broadcasted_iota(jnp.int32, (R, R), 0)
           == lax.broadcasted_iota(jnp.int32, (R, R), 1)).astype(jnp.float32)
    dn = (((0,), (0,)), ((), ()))
    ta = lax.dot_general(xa_ref[:], eye, dn,
                         preferred_element_type=jnp.float32)
    tb = lax.dot_general(xb_ref[:], eye, dn,
                         preferred_element_type=jnp.float32)
    o_ref[:] = jnp.concatenate([ta, tb], axis=1)


def _idx_prep_body(x_ref, o_ref):
    o_ref[:] = x_ref[:].reshape(3 * B)


def _cpd_sc_body(idx_hbm, t0_hbm, t1_hbm, t2_hbm, out_hbm,
                 idx_v, hidx_v, r0, r1, r2, out_v, sem):
    wid = lax.axis_index("s") * NC + lax.axis_index("c")
    base = wid * BPW

    for m in range(3):
        pltpu.sync_copy(idx_hbm.at[pl.ds(m * B + base, BPW)], idx_v.at[m])
    for m in range(3):
        for k in range(BPW // 16):
            iv = idx_v[m, pl.ds(k * 16, 16)]
            hidx_v[m, pl.ds(k * 16, 16)] = jnp.where(
                iv >= SPLIT, 2 * (iv - SPLIT) + 1, 2 * iv)

    copies = []
    for m, (tab, dst) in enumerate(((t0_hbm, r0), (t1_hbm, r1),
                                    (t2_hbm, r2))):
        for j in range(BPW // CH):
            copies.append(pltpu.async_copy(
                tab.at[hidx_v.at[m, pl.ds(j * CH, CH)]],
                dst.at[pl.ds(j * CH, CH)], sem))
    for cp in copies:
        cp.wait()

    def group(g, c2):
        b0 = g * 16
        for rr in range(16):
            row = b0 + rr
            acc = None
            for c in range(4):
                a = r0[row, pl.ds(c * 16, 16)]
                bb = r1[row, pl.ds(c * 16, 16)]
                d = r2[row, pl.ds(c * 16, 16)]
                pv = a * bb * d
                acc = pv if acc is None else acc + pv
            out_v[pl.ds(row * 16, 16)] = acc
        return c2

    lax.fori_loop(0, BPW // 16, group, 0)

    pltpu.sync_copy(out_v, out_hbm.at[pl.ds(wid * BPW * 16, BPW * 16)])


def _reduce_tc_body(x_ref, o_ref):
    x = x_ref[:].reshape(512, 128)
    l_i = lax.broadcasted_iota(jnp.int32, (128, 128), 0)
    j_i = lax.broadcasted_iota(jnp.int32, (128, 128), 1)
    wa = ((j_i < 64) & (l_i // 2 == j_i)).astype(jnp.float32)
    wb = ((j_i >= 64) & (l_i // 2 == j_i - 64)).astype(jnp.float32)
    n = 512
    for _ in range(4):
        h = n // 2
        r_i = lax.broadcasted_iota(jnp.int32, (h, n), 0)
        c_i = lax.broadcasted_iota(jnp.int32, (h, n), 1)
        ae = (c_i == 2 * r_i).astype(jnp.float32)
        ao = (c_i == 2 * r_i + 1).astype(jnp.float32)
        xe = jnp.dot(ae, x, preferred_element_type=jnp.float32)
        xo = jnp.dot(ao, x, preferred_element_type=jnp.float32)
        x = (jnp.dot(xe, wa, preferred_element_type=jnp.float32)
             + jnp.dot(xo, wb, preferred_element_type=jnp.float32))
        n = h
    o_ref[:] = x.reshape(4096)


def _prep_table(ev):
    return pl.pallas_call(
        _table_prep_body,
        grid=(GRID_T,),
        in_specs=[pl.BlockSpec((R, W), lambda g: (0, g)),
                  pl.BlockSpec((R, W),
                               lambda g: (0, jnp.minimum(g + GRID_T,
                                                         (V - 1) // W)))],
        out_specs=pl.BlockSpec((W, 2 * R), lambda g: (g, 0)),
        out_shape=jax.ShapeDtypeStruct((SPLIT, 2 * R), jnp.float32),
    )(ev, ev)


@jax.jit
def kernel(idxs, E0, E1, E2):
    idxs = idxs.astype(jnp.int32)

    idx_flat = pl.pallas_call(
        _idx_prep_body,
        in_specs=[pl.BlockSpec((3, B), lambda: (0, 0))],
        out_specs=pl.BlockSpec((3 * B,), lambda: (0,)),
        out_shape=jax.ShapeDtypeStruct((3 * B,), jnp.int32),
    )(jnp.transpose(idxs))

    t0 = _prep_table(jnp.transpose(E0)).reshape(2 * SPLIT, R)
    t1 = _prep_table(jnp.transpose(E1)).reshape(2 * SPLIT, R)
    t2 = _prep_table(jnp.transpose(E2)).reshape(2 * SPLIT, R)

    mesh = plsc.VectorSubcoreMesh(core_axis_name="c", subcore_axis_name="s")
    sc_fn = pl.kernel(
        _cpd_sc_body,
        mesh=mesh,
        out_type=jax.ShapeDtypeStruct((B * 16,), jnp.float32),
        scratch_types=[
            pltpu.VMEM((3, BPW), jnp.int32),
            pltpu.VMEM((3, BPW), jnp.int32),
            pltpu.VMEM((BPW, R), jnp.float32),
            pltpu.VMEM((BPW, R), jnp.float32),
            pltpu.VMEM((BPW, R), jnp.float32),
            pltpu.VMEM((BPW * 16,), jnp.float32),
            pltpu.SemaphoreType.DMA,
        ],
        compiler_params=pltpu.CompilerParams(use_tc_tiling_on_sc=False),
    )
    partials = sc_fn(idx_flat, t0, t1, t2)

    red_rows = 4096
    out = pl.pallas_call(
        _reduce_tc_body,
        grid=(B // red_rows,),
        in_specs=[pl.BlockSpec((red_rows * 16,), lambda i: (i,))],
        out_specs=pl.BlockSpec((red_rows,), lambda i: (i,)),
        out_shape=jax.ShapeDtypeStruct((B,), jnp.float32),
    )(partials)
    return out
